# Initial kernel scaffold; baseline (speedup 1.0000x reference)
import jax
import jax.numpy as jnp
from jax.experimental import pallas as pl


def kernel(x, edge_index, W1, b1, W2, b2):
    raise NotImplementedError("write your pallas kernel here")


# speedup vs baseline: 9.1091x; 9.1091x over previous
import functools

import jax
import jax.numpy as jnp
from jax import lax
from jax.experimental import pallas as pl
from jax.experimental.pallas import tpu as pltpu
from jax.experimental.pallas import tpu_sc as plsc

_N = 10000
_D = 128
_E = 320000
_NPAD = 10240
_NC = 2
_NS = 16
_NW = _NC * _NS
_CH = 128
_NCH = 80
_EPW = _CH * _NCH
_EPAD = _NW * _EPW
_STRIPE = _NPAD // _NS
_ZR = 64
_RB = 10


def _mesh():
    return plsc.VectorSubcoreMesh(core_axis_name="c", subcore_axis_name="s")


def _deg_body(src_hbm, dst_hbm, out_hbm, sidx, didx, ones, zblk, deg_s, deg_d):
    c = lax.axis_index("c")
    s = lax.axis_index("s")
    wid = s * _NC + c
    for j in range(_CH // 16):
        ones[pl.ds(j * 16, 16)] = jnp.ones((16,), jnp.float32)
    for j in range(_STRIPE // 16):
        zblk[pl.ds(j * 16, 16)] = jnp.zeros((16,), jnp.float32)
    pltpu.sync_copy(zblk, deg_s.at[pl.ds(s * _STRIPE, _STRIPE)])
    pltpu.sync_copy(zblk, deg_d.at[pl.ds(s * _STRIPE, _STRIPE)])
    pltpu.sync_copy(src_hbm.at[pl.ds(wid * _NCH, _NCH)], sidx)
    pltpu.sync_copy(dst_hbm.at[pl.ds(wid * _NCH, _NCH)], didx)
    plsc.subcore_barrier()

    def body(i, carry):
        pltpu.sync_copy(ones, deg_s.at[sidx.at[i]], add=True)
        pltpu.sync_copy(ones, deg_d.at[didx.at[i]], add=True)
        return carry

    lax.fori_loop(0, _NCH, body, 0)
    plsc.subcore_barrier()
    pltpu.sync_copy(deg_s.at[pl.ds(s * _STRIPE, _STRIPE)],
                    out_hbm.at[c, 0, pl.ds(s * _STRIPE, _STRIPE)])
    pltpu.sync_copy(deg_d.at[pl.ds(s * _STRIPE, _STRIPE)],
                    out_hbm.at[c, 1, pl.ds(s * _STRIPE, _STRIPE)])


def _sc_degrees(src2, dst2):
    k = pl.kernel(
        _deg_body,
        out_type=jax.ShapeDtypeStruct((_NC, 2, _NPAD), jnp.float32),
        mesh=_mesh(),
        scratch_types=[
            pltpu.VMEM((_NCH, _CH), jnp.int32),
            pltpu.VMEM((_NCH, _CH), jnp.int32),
            pltpu.VMEM((_CH,), jnp.float32),
            pltpu.VMEM((_STRIPE,), jnp.float32),
            pltpu.VMEM_SHARED((_NPAD,), jnp.float32),
            pltpu.VMEM_SHARED((_NPAD,), jnp.float32),
        ],
    )
    return k(src2, dst2)


def _agg_body(tab_hbm, src_hbm, dst_hbm, out_hbm, sidx, didx, rows, zblk, agg,
              semg):
    c = lax.axis_index("c")
    s = lax.axis_index("s")
    wid = s * _NC + c

    def zrow(r, carry):
        for j in range(_D // 16):
            zblk[r, pl.ds(j * 16, 16)] = jnp.zeros((16,), jnp.float32)
        return carry

    lax.fori_loop(0, _ZR, zrow, 0)

    def zcopy(kk, carry):
        pltpu.sync_copy(zblk, agg.at[pl.ds(s * _STRIPE + kk * _ZR, _ZR)])
        return carry

    lax.fori_loop(0, _RB, zcopy, 0)
    pltpu.sync_copy(src_hbm.at[pl.ds(wid * _NCH, _NCH)], sidx)
    pltpu.sync_copy(dst_hbm.at[pl.ds(wid * _NCH, _NCH)], didx)
    plsc.subcore_barrier()

    def body(i, carry):
        pltpu.async_copy(tab_hbm.at[sidx.at[i]], rows, semg).wait()
        pltpu.sync_copy(rows, agg.at[didx.at[i]], add=True)
        return carry

    lax.fori_loop(0, _NCH, body, 0)
    plsc.subcore_barrier()
    pltpu.sync_copy(agg.at[pl.ds(s * _STRIPE, _STRIPE)],
                    out_hbm.at[c, pl.ds(s * _STRIPE, _STRIPE)])


def _sc_agg(table, src2, dst2):
    k = pl.kernel(
        _agg_body,
        out_type=jax.ShapeDtypeStruct((_NC, _NPAD, _D), jnp.float32),
        mesh=_mesh(),
        scratch_types=[
            pltpu.VMEM((_NCH, _CH), jnp.int32),
            pltpu.VMEM((_NCH, _CH), jnp.int32),
            pltpu.VMEM((_CH, _D), jnp.float32),
            pltpu.VMEM((_ZR, _D), jnp.float32),
            pltpu.VMEM_SHARED((_NPAD, _D), jnp.float32),
            pltpu.SemaphoreType.DMA,
        ],
    )
    return k(table, src2, dst2)


_TB = 1024


def _prep_tc(x_pad, deg):
    def body(x_ref, deg_ref, xn_ref, ns_ref, nd_ref):
        dg = deg_ref[...]
        ns = lax.rsqrt(jnp.maximum(dg[0, 0] + dg[1, 0], 1.0))
        nd = lax.rsqrt(jnp.maximum(dg[0, 1] + dg[1, 1], 1.0))
        ns_ref[...] = ns
        nd_ref[...] = nd
        xn_ref[...] = x_ref[...] * ns[:, None]

    return pl.pallas_call(
        body,
        grid=(_NPAD // _TB,),
        in_specs=[
            pl.BlockSpec((_TB, _D), lambda i: (i, 0)),
            pl.BlockSpec((_NC, 2, _TB), lambda i: (0, 0, i)),
        ],
        out_specs=[
            pl.BlockSpec((_TB, _D), lambda i: (i, 0)),
            pl.BlockSpec((_TB,), lambda i: (i,)),
            pl.BlockSpec((_TB,), lambda i: (i,)),
        ],
        out_shape=[
            jax.ShapeDtypeStruct((_NPAD, _D), jnp.float32),
            jax.ShapeDtypeStruct((_NPAD,), jnp.float32),
            jax.ShapeDtypeStruct((_NPAD,), jnp.float32),
        ],
    )(x_pad, deg)


def _layer1_tc(p1, x_pad, ns, nd, W1, b1):
    def body(p_ref, x_ref, ns_ref, nd_ref, w_ref, b_ref, hn_ref):
        agg = p_ref[0] + p_ref[1]
        rst = agg * nd_ref[...][:, None]
        out1 = jnp.dot(rst, w_ref[...], preferred_element_type=jnp.float32)
        out1 = out1 + b_ref[...][None, :]
        h = jnp.maximum(out1, 0.0) + x_ref[...]
        hn_ref[...] = h * ns_ref[...][:, None]

    return pl.pallas_call(
        body,
        grid=(_NPAD // _TB,),
        in_specs=[
            pl.BlockSpec((_NC, _TB, _D), lambda i: (0, i, 0)),
            pl.BlockSpec((_TB, _D), lambda i: (i, 0)),
            pl.BlockSpec((_TB,), lambda i: (i,)),
            pl.BlockSpec((_TB,), lambda i: (i,)),
            pl.BlockSpec((_D, _D), lambda i: (0, 0)),
            pl.BlockSpec((_D,), lambda i: (0,)),
        ],
        out_specs=pl.BlockSpec((_TB, _D), lambda i: (i, 0)),
        out_shape=jax.ShapeDtypeStruct((_NPAD, _D), jnp.float32),
    )(p1, x_pad, ns, nd, W1, b1)


def _layer2_tc(p2, nd, W2, b2):
    def body(p_ref, nd_ref, w_ref, b_ref, out_ref):
        agg = p_ref[0] + p_ref[1]
        rst = agg * nd_ref[...][:, None]
        out = jnp.dot(rst, w_ref[...], preferred_element_type=jnp.float32)
        out_ref[...] = out + b_ref[...][None, :]

    return pl.pallas_call(
        body,
        grid=(_NPAD // _TB,),
        in_specs=[
            pl.BlockSpec((_NC, _TB, _D), lambda i: (0, i, 0)),
            pl.BlockSpec((_TB,), lambda i: (i,)),
            pl.BlockSpec((_D, _D), lambda i: (0, 0)),
            pl.BlockSpec((_D,), lambda i: (0,)),
        ],
        out_specs=pl.BlockSpec((_TB, _D), lambda i: (i, 0)),
        out_shape=jax.ShapeDtypeStruct((_NPAD, _D), jnp.float32),
    )(p2, nd, W2, b2)


def kernel(x, edge_index, W1, b1, W2, b2):
    x_pad = jnp.pad(x, ((0, _NPAD - _N), (0, 0)))
    npad_e = _EPAD - _E
    dummy = _N + (jnp.arange(npad_e, dtype=jnp.int32) % (_NPAD - _N))
    src = jnp.concatenate([edge_index[0], dummy])
    dst = jnp.concatenate([edge_index[1], dummy])
    src2 = src.reshape(_EPAD // _CH, _CH)
    dst2 = dst.reshape(_EPAD // _CH, _CH)

    deg = _sc_degrees(src2, dst2)
    xn, ns, nd = _prep_tc(x_pad, deg)
    p1 = _sc_agg(xn, src2, dst2)
    hn = _layer1_tc(p1, x_pad, ns, nd, W1, b1)
    p2 = _sc_agg(hn, src2, dst2)
    out = _layer2_tc(p2, nd, W2, b2)
    return out[:_N]


# speedup vs baseline: 13.3245x; 1.4628x over previous
import functools

import jax
import jax.numpy as jnp
from jax import lax
from jax.experimental import pallas as pl
from jax.experimental.pallas import tpu as pltpu
from jax.experimental.pallas import tpu_sc as plsc

_N = 10000
_D = 128
_E = 320000
_NPAD = 10240
_NC = 2
_NS = 16
_NW = _NC * _NS
_CH = 128
_NCH = 80
_EPW = _CH * _NCH
_EPAD = _NW * _EPW
_STRIPE = _NPAD // _NS


def _mesh():
    return plsc.VectorSubcoreMesh(core_axis_name="c", subcore_axis_name="s")


def _unpack_chunk(pidx, ch, sdst, b):
    for j in range(_D // 16):
        p = pidx[ch, pl.ds(j * 16, 16)]
        sdst[0][b, pl.ds(j * 16, 16)] = p & jnp.int32(0xFFFF)
        sdst[1][b, pl.ds(j * 16, 16)] = jax.lax.shift_right_logical(
            p, jnp.int32(16))


def _deg_body(pk_hbm, out_hbm, pidx, sidx, didx, ones, zblk, deg_s, deg_d):
    c = lax.axis_index("c")
    s = lax.axis_index("s")
    wid = s * _NC + c
    for j in range(_CH // 16):
        ones[pl.ds(j * 16, 16)] = jnp.ones((16,), jnp.float32)
    for j in range(_STRIPE // 16):
        zblk[pl.ds(j * 16, 16)] = jnp.zeros((16,), jnp.float32)
    pltpu.sync_copy(zblk, deg_s.at[pl.ds(s * _STRIPE, _STRIPE)])
    pltpu.sync_copy(zblk, deg_d.at[pl.ds(s * _STRIPE, _STRIPE)])
    pltpu.sync_copy(pk_hbm.at[pl.ds(wid * _NCH, _NCH)], pidx)
    plsc.subcore_barrier()

    def body(i, carry):
        _unpack_chunk(pidx, i, (sidx, didx), 0)
        pltpu.sync_copy(ones, deg_s.at[sidx.at[0]], add=True)
        pltpu.sync_copy(ones, deg_d.at[didx.at[0]], add=True)
        return carry

    lax.fori_loop(0, _NCH, body, 0)
    plsc.subcore_barrier()
    pltpu.sync_copy(deg_s.at[pl.ds(s * _STRIPE, _STRIPE)],
                    out_hbm.at[c, 0, pl.ds(s * _STRIPE, _STRIPE)])
    pltpu.sync_copy(deg_d.at[pl.ds(s * _STRIPE, _STRIPE)],
                    out_hbm.at[c, 1, pl.ds(s * _STRIPE, _STRIPE)])


def _sc_degrees(pk2):
    k = pl.kernel(
        _deg_body,
        out_type=jax.ShapeDtypeStruct((_NC, 2, _NPAD), jnp.float32),
        mesh=_mesh(),
        scratch_types=[
            pltpu.VMEM((_NCH, _CH), jnp.int32),
            pltpu.VMEM((1, _CH), jnp.int32),
            pltpu.VMEM((1, _CH), jnp.int32),
            pltpu.VMEM((_CH,), jnp.float32),
            pltpu.VMEM((_STRIPE,), jnp.float32),
            pltpu.VMEM_SHARED((_NPAD,), jnp.float32),
            pltpu.VMEM_SHARED((_NPAD,), jnp.float32),
        ],
    )
    return k(pk2)


def _agg_body(tab_hbm, pk_hbm, out_hbm, pidx, sidx, didx, rows, agg,
              sem0, sem1):
    c = lax.axis_index("c")
    s = lax.axis_index("s")
    wid = s * _NC + c

    def zrow(r, carry):
        for j in range(_D // 16):
            rows[0, r, pl.ds(j * 16, 16)] = jnp.zeros((16,), jnp.float32)
        return carry

    lax.fori_loop(0, _CH, zrow, 0)

    def zcopy(kk, carry):
        pltpu.sync_copy(rows.at[0], agg.at[pl.ds(s * _STRIPE + kk * _CH, _CH)])
        return carry

    lax.fori_loop(0, _STRIPE // _CH, zcopy, 0)
    pltpu.sync_copy(pk_hbm.at[pl.ds(wid * _NCH, _NCH)], pidx)
    plsc.subcore_barrier()

    sems = (sem0, sem1)

    def gstart(ch, b):
        pltpu.async_copy(tab_hbm.at[sidx.at[b]], rows.at[b], sems[b])

    def gwait(b):
        pltpu.make_async_copy(tab_hbm.at[sidx.at[b]], rows.at[b],
                              sems[b]).wait()

    for b in range(2):
        _unpack_chunk(pidx, b, (sidx, didx), b)
        gstart(b, b)

    def body(i, carry):
        for b in range(2):
            ch = i * 2 + b
            gwait(b)
            pltpu.sync_copy(rows.at[b], agg.at[didx.at[b]], add=True)
            _unpack_chunk(pidx, ch + 2, (sidx, didx), b)
            gstart(ch + 2, b)
        return carry

    lax.fori_loop(0, _NCH // 2 - 1, body, 0)
    for b in range(2):
        gwait(b)
        pltpu.sync_copy(rows.at[b], agg.at[didx.at[b]], add=True)
    plsc.subcore_barrier()
    pltpu.sync_copy(agg.at[pl.ds(s * _STRIPE, _STRIPE)],
                    out_hbm.at[c, pl.ds(s * _STRIPE, _STRIPE)])


def _sc_agg(table, pk2):
    k = pl.kernel(
        _agg_body,
        out_type=jax.ShapeDtypeStruct((_NC, _NPAD, _D), jnp.float32),
        mesh=_mesh(),
        scratch_types=[
            pltpu.VMEM((_NCH, _CH), jnp.int32),
            pltpu.VMEM((2, _CH), jnp.int32),
            pltpu.VMEM((2, _CH), jnp.int32),
            pltpu.VMEM((2, _CH, _D), jnp.float32),
            pltpu.VMEM_SHARED((_NPAD, _D), jnp.float32),
            pltpu.SemaphoreType.DMA,
            pltpu.SemaphoreType.DMA,
        ],
    )
    return k(table, pk2)


_TB = 1024


def _prep_tc(x_pad, deg):
    def body(x_ref, deg_ref, xn_ref, ns_ref, nd_ref):
        dg = deg_ref[...]
        ns = lax.rsqrt(jnp.maximum(dg[0, 0] + dg[1, 0], 1.0))
        nd = lax.rsqrt(jnp.maximum(dg[0, 1] + dg[1, 1], 1.0))
        ns_ref[...] = ns
        nd_ref[...] = nd
        xn_ref[...] = x_ref[...] * ns[:, None]

    return pl.pallas_call(
        body,
        grid=(_NPAD // _TB,),
        in_specs=[
            pl.BlockSpec((_TB, _D), lambda i: (i, 0)),
            pl.BlockSpec((_NC, 2, _TB), lambda i: (0, 0, i)),
        ],
        out_specs=[
            pl.BlockSpec((_TB, _D), lambda i: (i, 0)),
            pl.BlockSpec((_TB,), lambda i: (i,)),
            pl.BlockSpec((_TB,), lambda i: (i,)),
        ],
        out_shape=[
            jax.ShapeDtypeStruct((_NPAD, _D), jnp.float32),
            jax.ShapeDtypeStruct((_NPAD,), jnp.float32),
            jax.ShapeDtypeStruct((_NPAD,), jnp.float32),
        ],
    )(x_pad, deg)


def _layer1_tc(p1, x_pad, ns, nd, W1, b1):
    def body(p_ref, x_ref, ns_ref, nd_ref, w_ref, b_ref, hn_ref):
        agg = p_ref[0] + p_ref[1]
        rst = agg * nd_ref[...][:, None]
        out1 = jnp.dot(rst, w_ref[...], preferred_element_type=jnp.float32)
        out1 = out1 + b_ref[...][None, :]
        h = jnp.maximum(out1, 0.0) + x_ref[...]
        hn_ref[...] = h * ns_ref[...][:, None]

    return pl.pallas_call(
        body,
        grid=(_NPAD // _TB,),
        in_specs=[
            pl.BlockSpec((_NC, _TB, _D), lambda i: (0, i, 0)),
            pl.BlockSpec((_TB, _D), lambda i: (i, 0)),
            pl.BlockSpec((_TB,), lambda i: (i,)),
            pl.BlockSpec((_TB,), lambda i: (i,)),
            pl.BlockSpec((_D, _D), lambda i: (0, 0)),
            pl.BlockSpec((_D,), lambda i: (0,)),
        ],
        out_specs=pl.BlockSpec((_TB, _D), lambda i: (i, 0)),
        out_shape=jax.ShapeDtypeStruct((_NPAD, _D), jnp.float32),
    )(p1, x_pad, ns, nd, W1, b1)


def _layer2_tc(p2, nd, W2, b2):
    def body(p_ref, nd_ref, w_ref, b_ref, out_ref):
        agg = p_ref[0] + p_ref[1]
        rst = agg * nd_ref[...][:, None]
        out = jnp.dot(rst, w_ref[...], preferred_element_type=jnp.float32)
        out_ref[...] = out + b_ref[...][None, :]

    return pl.pallas_call(
        body,
        grid=(_NPAD // _TB,),
        in_specs=[
            pl.BlockSpec((_NC, _TB, _D), lambda i: (0, i, 0)),
            pl.BlockSpec((_TB,), lambda i: (i,)),
            pl.BlockSpec((_D, _D), lambda i: (0, 0)),
            pl.BlockSpec((_D,), lambda i: (0,)),
        ],
        out_specs=pl.BlockSpec((_TB, _D), lambda i: (i, 0)),
        out_shape=jax.ShapeDtypeStruct((_NPAD, _D), jnp.float32),
    )(p2, nd, W2, b2)


def kernel(x, edge_index, W1, b1, W2, b2):
    x_pad = jnp.pad(x, ((0, _NPAD - _N), (0, 0)))
    npad_e = _EPAD - _E
    dummy = _N + (jnp.arange(npad_e, dtype=jnp.int32) % (_NPAD - _N))
    src = jnp.concatenate([edge_index[0], dummy])
    dst = jnp.concatenate([edge_index[1], dummy])
    packed = src | (dst << jnp.int32(16))
    pk2 = packed.reshape(_EPAD // _CH, _CH)

    deg = _sc_degrees(pk2)
    xn, ns, nd = _prep_tc(x_pad, deg)
    p1 = _sc_agg(xn, pk2)
    hn = _layer1_tc(p1, x_pad, ns, nd, W1, b1)
    p2 = _sc_agg(hn, pk2)
    out = _layer2_tc(p2, nd, W2, b2)
    return out[:_N]
